# baseline (speedup 1.0000x reference)
import functools

import jax
import jax.numpy as jnp
from jax import lax
from jax.experimental import pallas as pl
from jax.experimental.pallas import tpu as pltpu
from jax.experimental.pallas import tpu_sc as plsc

N = 10000
E = 320000
D = 128
H = 8
DH = 16
D_FF = 256

NC = 2
NS = 16
NW = NC * NS

NPAD = 10240
EPW = 10240
EPAD = NW * EPW
C = 32
NCHUNK = EPW // C
ROWS_PER_SUB = NPAD // NS
ACC_W = D + H


def _dg(a, b):
    return lax.dot_general(a, b, (((1,), (1,)), ((), ())),
                           preferred_element_type=jnp.float32)


def _qkv_body(x_ref, wq_ref, wk_ref, wv_ref, q_ref, kv_ref):
    x = x_ref[:]
    q_ref[:] = _dg(x, wq_ref[:])
    kv_ref[:, :D] = _dg(x, wk_ref[:])
    kv_ref[:, D:] = _dg(x, wv_ref[:])


def _edge_body(q_hbm, kv_hbm, pk_hbm, out_hbm,
               pk_all,
               sidxA, tidxA, tshA, qbA, kbA, msgA,
               sidxB, tidxB, tshB, qbB, kbB, msgB,
               acc,
               semqA, semkA, semvA, semsA, semqB, semkB, semvB, semsB):
    cid = lax.axis_index("c")
    sid = lax.axis_index("s")
    wid = sid * NC + cid

    zeros16 = jnp.zeros((16,), jnp.float32)

    def zero_msg(m):
        def zrow(g, _):
            eids = lax.iota(jnp.int32, 16) + g * 16

            def zf(f, _):
                plsc.store_scatter(m, [eids, jnp.zeros((16,), jnp.int32) + f], zeros16)
                return 0

            lax.fori_loop(0, ACC_W, zf, 0)
            return 0

        lax.fori_loop(0, C // 16, zrow, 0)

    zero_msg(msgA)
    zero_msg(msgB)

    def zcp(i, _):
        pltpu.sync_copy(msgA, acc.at[pl.ds(sid * ROWS_PER_SUB + i * C, C)])
        return 0

    lax.fori_loop(0, ROWS_PER_SUB // C, zcp, 0)

    pltpu.sync_copy(pk_hbm.at[pl.ds(wid * EPW, EPW)], pk_all.at[pl.ds(0, EPW)])
    dummy = jnp.full((16,), N + (N << 16), jnp.int32)
    dummyn = jnp.full((16,), N, jnp.int32)
    for j in range(2 * C // 16):
        pk_all[pl.ds(EPW + j * 16, 16)] = dummy
    for j in range(C // 16):
        tshA[pl.ds(j * 16, 16)] = dummyn
        tshB[pl.ds(j * 16, 16)] = dummyn
    plsc.subcore_barrier()

    def extract_idx(ci, sidx, tidx):
        for j in range(C // 16):
            v = pk_all[pl.ds(ci * C + j * 16, 16)]
            sidx[pl.ds(j * 16, 16)] = jnp.bitwise_and(v, 0xFFFF)
            tidx[pl.ds(j * 16, 16)] = lax.shift_right_logical(v, 16)

    def gathers(sidx, tidx, qb, kb, vb, semq, semk, semv):
        del vb, semv
        return (pltpu.make_async_copy(q_hbm.at[tidx], qb, semq),
                pltpu.make_async_copy(kv_hbm.at[sidx], kb, semk))

    gathersA = lambda: gathers(sidxA, tidxA, qbA, kbA, None, semqA, semkA, None)
    gathersB = lambda: gathers(sidxB, tidxB, qbB, kbB, None, semqB, semkB, None)
    scatterA = lambda: pltpu.make_async_copy(msgA, acc.at[tshA], semsA)
    scatterB = lambda: pltpu.make_async_copy(msgB, acc.at[tshB], semsB)

    def compute(qb, kb, vb, msg):
        def grp(g, _):
            eids = lax.iota(jnp.int32, 16) + g * 16
            lane = lax.iota(jnp.int32, 16)

            def hbody(h, _):
                fbase = jnp.zeros((16,), jnp.int32) + h * DH
                hl = lane + h
                accs = [None] * 4
                for r in range(DH):
                    c = fbase + jnp.bitwise_and(hl + r, 15)
                    p = plsc.load_gather(qb, [eids, c]) * plsc.load_gather(kb, [eids, c])
                    accs[r % 4] = p if accs[r % 4] is None else accs[r % 4] + p
                s = (accs[0] + accs[1]) + (accs[2] + accs[3])
                w = jnp.exp(s * 0.25)
                plsc.store_scatter(msg, [eids, jnp.zeros((16,), jnp.int32) + (D + h)], w)
                for r in range(DH):
                    c = fbase + jnp.bitwise_and(hl - r, 15)
                    vv = plsc.load_gather(vb, [eids, c])
                    plsc.store_scatter(msg, [eids, c], w * vv)
                return 0

            lax.fori_loop(0, H, hbody, 0)
            return 0

        lax.fori_loop(0, C // 16, grp, 0)

    def shadow(tidx, tsh):
        for j in range(C // 16):
            tsh[pl.ds(j * 16, 16)] = tidx[pl.ds(j * 16, 16)]

    scatterA().start(add=True)
    scatterB().start(add=True)
    extract_idx(0, sidxA, tidxA)
    for g in gathersA():
        g.start()

    def body(i2, _):
        cA = i2 * 2
        extract_idx(cA + 1, sidxB, tidxB)
        for g in gathersB():
            g.start()
        for g in gathersA():
            g.wait()
        scatterA().wait()
        shadow(tidxA, tshA)
        scatterA().start(add=True)
        extract_idx(cA + 2, sidxA, tidxA)
        for g in gathersA():
            g.start()
        for g in gathersB():
            g.wait()
        scatterB().wait()
        shadow(tidxB, tshB)
        scatterB().start(add=True)
        return 0

    lax.fori_loop(0, NCHUNK // 2, body, 0)
    for g in gathersA():
        g.wait()
    scatterA().wait()
    scatterB().wait()
    plsc.subcore_barrier()
    pltpu.sync_copy(acc.at[pl.ds(sid * ROWS_PER_SUB, ROWS_PER_SUB)],
                    out_hbm.at[cid, pl.ds(sid * ROWS_PER_SUB, ROWS_PER_SUB)])


def _ln(v, g, b):
    mu = jnp.mean(v, axis=-1, keepdims=True)
    var = jnp.mean((v - mu) ** 2, axis=-1, keepdims=True)
    return (v - mu) / jnp.sqrt(var + 1e-5) * g + b


def _post_body(a0_ref, a1_ref, x_ref, sel_ref, wo_ref, bo_ref, g1_ref, be1_ref,
               w1_ref, bb1_ref, w2_ref, bb2_ref, g2_ref, be2_ref, o_ref):
    a0 = a0_ref[:]
    a1 = a1_ref[:]
    num = a0[:, :D] + a1[:, :D]
    den = a0[:, D:] + a1[:, D:]
    den = jnp.where(den == 0.0, 1.0, den)
    denf = lax.dot_general(den, sel_ref[:], (((1,), (0,)), ((), ())),
                           preferred_element_type=jnp.float32)
    att = num / denf
    x = x_ref[:]
    y = _dg(att, wo_ref[:]) + bo_ref[:] + x
    y = _ln(y, g1_ref[:], be1_ref[:])
    h1 = jnp.maximum(_dg(y, w1_ref[:]) + bb1_ref[:], 0.0)
    z = _dg(h1, w2_ref[:]) + bb2_ref[:] + y
    o_ref[:] = _ln(z, g2_ref[:], be2_ref[:])


def kernel(node_feat, edge_index, WQ, WK, WV, WO, bO, ln1_g, ln1_b,
           W1, b1, W2, b2, ln2_g, ln2_b):
    xpad = jnp.pad(node_feat, ((0, NPAD - N), (0, 0)))
    src = jnp.pad(edge_index[0].astype(jnp.int32), (0, EPAD - E), constant_values=N)
    tgt = jnp.pad(edge_index[1].astype(jnp.int32), (0, EPAD - E), constant_values=N)
    packed = jnp.bitwise_or(src, jnp.left_shift(tgt, 16))

    BN = 256
    w_spec = pl.BlockSpec((D, D), lambda i: (0, 0))
    qkv = pl.pallas_call(
        _qkv_body,
        grid=(NPAD // BN,),
        in_specs=[pl.BlockSpec((BN, D), lambda i: (i, 0)), w_spec, w_spec, w_spec],
        out_specs=[pl.BlockSpec((BN, D), lambda i: (i, 0)),
                   pl.BlockSpec((BN, 2 * D), lambda i: (i, 0))],
        out_shape=[jax.ShapeDtypeStruct((NPAD, D), jnp.float32),
                   jax.ShapeDtypeStruct((NPAD, 2 * D), jnp.float32)],
    )
    q, kv = qkv(xpad, WQ, WK, WV)

    mesh = plsc.VectorSubcoreMesh(core_axis_name="c", subcore_axis_name="s",
                                  num_cores=NC, num_subcores=NS)
    half = [
        pltpu.VMEM((C,), jnp.int32),
        pltpu.VMEM((C,), jnp.int32),
        pltpu.VMEM((C,), jnp.int32),
        pltpu.VMEM((C, D), jnp.float32),
        pltpu.VMEM((C, 2 * D), jnp.float32),
        pltpu.VMEM((C, ACC_W), jnp.float32),
    ]
    edge_fn = functools.partial(
        pl.kernel,
        mesh=mesh,
        compiler_params=pltpu.CompilerParams(use_tc_tiling_on_sc=False,
                                             needs_layout_passes=False),
        out_type=jax.ShapeDtypeStruct((NC, NPAD, ACC_W), jnp.float32),
        scratch_types=(
            [pltpu.VMEM((EPW + 2 * C,), jnp.int32)]
            + half + half
            + [pltpu.VMEM_SHARED((NPAD, ACC_W), jnp.float32)]
            + [pltpu.SemaphoreType.DMA] * 8
        ),
    )(_edge_body)
    accs = edge_fn(q, kv, packed)

    sel = (jnp.arange(D, dtype=jnp.int32)[None, :] // DH
           == jnp.arange(H, dtype=jnp.int32)[:, None]).astype(jnp.float32)
    BM = 400
    full = lambda r, c: pl.BlockSpec((r, c), lambda i: (0, 0))
    out = pl.pallas_call(
        _post_body,
        grid=(N // BM,),
        in_specs=[
            pl.BlockSpec((BM, ACC_W), lambda i: (i, 0)),
            pl.BlockSpec((BM, ACC_W), lambda i: (i, 0)),
            pl.BlockSpec((BM, D), lambda i: (i, 0)),
            full(H, D),
            full(D, D),
            full(1, D),
            full(1, D),
            full(1, D),
            full(D_FF, D),
            full(1, D_FF),
            full(D, D_FF),
            full(1, D),
            full(1, D),
            full(1, D),
        ],
        out_specs=pl.BlockSpec((BM, D), lambda i: (i, 0)),
        out_shape=jax.ShapeDtypeStruct((N, D), jnp.float32),
    )(
        accs[0, :N], accs[1, :N], node_feat, sel, WO, bO.reshape(1, D),
        ln1_g.reshape(1, D), ln1_b.reshape(1, D), W1, b1.reshape(1, D_FF),
        W2, b2.reshape(1, D), ln2_g.reshape(1, D), ln2_b.reshape(1, D),
    )
    return out


# speedup vs baseline: 1.0912x; 1.0912x over previous
import functools

import jax
import jax.numpy as jnp
from jax import lax
from jax.experimental import pallas as pl
from jax.experimental.pallas import tpu as pltpu
from jax.experimental.pallas import tpu_sc as plsc

N = 10000
E = 320000
D = 128
H = 8
DH = 16
D_FF = 256

NC = 2
NS = 16
NW = NC * NS

NPAD = 10240
EPW = 10240
EPAD = NW * EPW
C = 32
NCHUNK = EPW // C
ROWS_PER_SUB = NPAD // NS
ACC_W = D + H


def _dg(a, b):
    return lax.dot_general(a, b, (((1,), (1,)), ((), ())),
                           preferred_element_type=jnp.float32)


def _qkv_body(x_ref, wq_ref, wk_ref, wv_ref, q_ref, k_ref, v_ref):
    x = x_ref[:]
    q_ref[:] = _dg(x, wq_ref[:]).astype(jnp.bfloat16)
    k_ref[:] = _dg(x, wk_ref[:]).astype(jnp.bfloat16)
    v_ref[:] = _dg(x, wv_ref[:]).astype(jnp.bfloat16)


def _edge_body(q_hbm, k_hbm, v_hbm, pk_hbm, out_hbm,
               pk_all,
               sidxA, tidxA, tshA, qbA, kbA, vbA, msgA,
               sidxB, tidxB, tshB, qbB, kbB, vbB, msgB,
               acc,
               semqA, semkA, semvA, semsA, semqB, semkB, semvB, semsB):
    cid = lax.axis_index("c")
    sid = lax.axis_index("s")
    wid = sid * NC + cid

    zeros16 = jnp.zeros((16,), jnp.float32)

    def zero_msg(m):
        def zrow(g, _):
            eids = lax.iota(jnp.int32, 16) + g * 16

            def zf(f, _):
                plsc.store_scatter(m, [eids, jnp.zeros((16,), jnp.int32) + f], zeros16)
                return 0

            lax.fori_loop(0, ACC_W, zf, 0)
            return 0

        lax.fori_loop(0, C // 16, zrow, 0)

    zero_msg(msgA)
    zero_msg(msgB)

    def zcp(i, _):
        pltpu.sync_copy(msgA, acc.at[pl.ds(sid * ROWS_PER_SUB + i * C, C)])
        return 0

    lax.fori_loop(0, ROWS_PER_SUB // C, zcp, 0)

    pltpu.sync_copy(pk_hbm.at[pl.ds(wid * EPW, EPW)], pk_all.at[pl.ds(0, EPW)])
    dummy = jnp.full((16,), N + (N << 16), jnp.int32)
    dummyn = jnp.full((16,), N, jnp.int32)
    for j in range(2 * C // 16):
        pk_all[pl.ds(EPW + j * 16, 16)] = dummy
    for j in range(C // 16):
        tshA[pl.ds(j * 16, 16)] = dummyn
        tshB[pl.ds(j * 16, 16)] = dummyn
    plsc.subcore_barrier()

    def extract_idx(ci, sidx, tidx):
        for j in range(C // 16):
            v = pk_all[pl.ds(ci * C + j * 16, 16)]
            sidx[pl.ds(j * 16, 16)] = jnp.bitwise_and(v, 0xFFFF)
            tidx[pl.ds(j * 16, 16)] = lax.shift_right_logical(v, 16)

    def gathers(sidx, tidx, qb, kb, vb, semq, semk, semv):
        return (pltpu.make_async_copy(q_hbm.at[tidx], qb, semq),
                pltpu.make_async_copy(k_hbm.at[sidx], kb, semk),
                pltpu.make_async_copy(v_hbm.at[sidx], vb, semv))

    gathersA = lambda: gathers(sidxA, tidxA, qbA, kbA, vbA, semqA, semkA, semvA)
    gathersB = lambda: gathers(sidxB, tidxB, qbB, kbB, vbB, semqB, semkB, semvB)
    scatterA = lambda: pltpu.make_async_copy(msgA, acc.at[tshA], semsA)
    scatterB = lambda: pltpu.make_async_copy(msgB, acc.at[tshB], semsB)

    def compute(qb, kb, vb, msg):
        def grp(g, _):
            eids = lax.iota(jnp.int32, 16) + g * 16
            lane = lax.iota(jnp.int32, 16)

            def hbody(h, _):
                pbase = jnp.zeros((16,), jnp.int32) + h * (DH // 2)
                hl = lane + h
                ilv = plsc.PackFormat.INTERLEAVED
                acc0 = None
                acc1 = None
                for r in range(DH // 2):
                    c = pbase + jnp.bitwise_and(hl + r, 7)
                    qi = plsc.load_gather(qb, [eids, c])
                    ki = plsc.load_gather(kb, [eids, c])
                    pb = plsc.bitcast(qi, jnp.bfloat16) * plsc.bitcast(ki, jnp.bfloat16)
                    plo, phi = plsc.unpack(pb, format=ilv)
                    acc0 = plo if acc0 is None else acc0 + plo
                    acc1 = phi if acc1 is None else acc1 + phi
                s = acc0 + acc1
                w = jnp.exp(s * 0.25)
                plsc.store_scatter(msg, [eids, jnp.zeros((16,), jnp.int32) + (D + h)], w)
                for r in range(DH // 2):
                    c = pbase + jnp.bitwise_and(hl - r, 7)
                    vi = plsc.load_gather(vb, [eids, c])
                    vlo, vhi = plsc.unpack(plsc.bitcast(vi, jnp.bfloat16), format=ilv)
                    f0 = c + c
                    plsc.store_scatter(msg, [eids, f0], w * vlo)
                    plsc.store_scatter(msg, [eids, f0 + 1], w * vhi)
                return 0

            lax.fori_loop(0, H, hbody, 0)
            return 0

        lax.fori_loop(0, C // 16, grp, 0)

    def shadow(tidx, tsh):
        for j in range(C // 16):
            tsh[pl.ds(j * 16, 16)] = tidx[pl.ds(j * 16, 16)]

    scatterA().start(add=True)
    scatterB().start(add=True)
    extract_idx(0, sidxA, tidxA)
    for g in gathersA():
        g.start()

    def body(i2, _):
        cA = i2 * 2
        extract_idx(cA + 1, sidxB, tidxB)
        for g in gathersB():
            g.start()
        for g in gathersA():
            g.wait()
        scatterA().wait()
        compute(qbA, kbA, vbA, msgA)
        shadow(tidxA, tshA)
        scatterA().start(add=True)
        extract_idx(cA + 2, sidxA, tidxA)
        for g in gathersA():
            g.start()
        for g in gathersB():
            g.wait()
        scatterB().wait()
        compute(qbB, kbB, vbB, msgB)
        shadow(tidxB, tshB)
        scatterB().start(add=True)
        return 0

    lax.fori_loop(0, NCHUNK // 2, body, 0)
    for g in gathersA():
        g.wait()
    scatterA().wait()
    scatterB().wait()
    plsc.subcore_barrier()
    pltpu.sync_copy(acc.at[pl.ds(sid * ROWS_PER_SUB, ROWS_PER_SUB)],
                    out_hbm.at[cid, pl.ds(sid * ROWS_PER_SUB, ROWS_PER_SUB)])


def _ln(v, g, b):
    mu = jnp.mean(v, axis=-1, keepdims=True)
    var = jnp.mean((v - mu) ** 2, axis=-1, keepdims=True)
    return (v - mu) / jnp.sqrt(var + 1e-5) * g + b


def _post_body(a0_ref, a1_ref, x_ref, sel_ref, wo_ref, bo_ref, g1_ref, be1_ref,
               w1_ref, bb1_ref, w2_ref, bb2_ref, g2_ref, be2_ref, o_ref):
    a0 = a0_ref[:]
    a1 = a1_ref[:]
    num = a0[:, :D] + a1[:, :D]
    den = a0[:, D:] + a1[:, D:]
    den = jnp.where(den == 0.0, 1.0, den)
    denf = lax.dot_general(den, sel_ref[:], (((1,), (0,)), ((), ())),
                           preferred_element_type=jnp.float32)
    att = num / denf
    x = x_ref[:]
    y = _dg(att, wo_ref[:]) + bo_ref[:] + x
    y = _ln(y, g1_ref[:], be1_ref[:])
    h1 = jnp.maximum(_dg(y, w1_ref[:]) + bb1_ref[:], 0.0)
    z = _dg(h1, w2_ref[:]) + bb2_ref[:] + y
    o_ref[:] = _ln(z, g2_ref[:], be2_ref[:])


def kernel(node_feat, edge_index, WQ, WK, WV, WO, bO, ln1_g, ln1_b,
           W1, b1, W2, b2, ln2_g, ln2_b):
    xpad = jnp.pad(node_feat, ((0, NPAD - N), (0, 0)))
    src = jnp.pad(edge_index[0].astype(jnp.int32), (0, EPAD - E), constant_values=N)
    tgt = jnp.pad(edge_index[1].astype(jnp.int32), (0, EPAD - E), constant_values=N)
    packed = jnp.bitwise_or(src, jnp.left_shift(tgt, 16))

    BN = 256
    w_spec = pl.BlockSpec((D, D), lambda i: (0, 0))
    qkv = pl.pallas_call(
        _qkv_body,
        grid=(NPAD // BN,),
        in_specs=[pl.BlockSpec((BN, D), lambda i: (i, 0)), w_spec, w_spec, w_spec],
        out_specs=[pl.BlockSpec((BN, D), lambda i: (i, 0))] * 3,
        out_shape=[jax.ShapeDtypeStruct((NPAD, D), jnp.bfloat16)] * 3,
    )
    q, k, v = qkv(xpad, WQ, WK, WV)
    def _p32(a):
        return lax.bitcast_convert_type(a.reshape(NPAD, D // 2, 2), jnp.int32)
    q, k, v = _p32(q), _p32(k), _p32(v)

    mesh = plsc.VectorSubcoreMesh(core_axis_name="c", subcore_axis_name="s",
                                  num_cores=NC, num_subcores=NS)
    half = [
        pltpu.VMEM((C,), jnp.int32),
        pltpu.VMEM((C,), jnp.int32),
        pltpu.VMEM((C,), jnp.int32),
        pltpu.VMEM((C, D // 2), jnp.int32),
        pltpu.VMEM((C, D // 2), jnp.int32),
        pltpu.VMEM((C, D // 2), jnp.int32),
        pltpu.VMEM((C, ACC_W), jnp.float32),
    ]
    edge_fn = functools.partial(
        pl.kernel,
        mesh=mesh,
        compiler_params=pltpu.CompilerParams(use_tc_tiling_on_sc=False,
                                             needs_layout_passes=False),
        out_type=jax.ShapeDtypeStruct((NC, NPAD, ACC_W), jnp.float32),
        scratch_types=(
            [pltpu.VMEM((EPW + 2 * C,), jnp.int32)]
            + half + half
            + [pltpu.VMEM_SHARED((NPAD, ACC_W), jnp.float32)]
            + [pltpu.SemaphoreType.DMA] * 8
        ),
    )(_edge_body)
    accs = edge_fn(q, k, v, packed)

    sel = (jnp.arange(D, dtype=jnp.int32)[None, :] // DH
           == jnp.arange(H, dtype=jnp.int32)[:, None]).astype(jnp.float32)
    BM = 400
    full = lambda r, c: pl.BlockSpec((r, c), lambda i: (0, 0))
    out = pl.pallas_call(
        _post_body,
        grid=(N // BM,),
        in_specs=[
            pl.BlockSpec((BM, ACC_W), lambda i: (i, 0)),
            pl.BlockSpec((BM, ACC_W), lambda i: (i, 0)),
            pl.BlockSpec((BM, D), lambda i: (i, 0)),
            full(H, D),
            full(D, D),
            full(1, D),
            full(1, D),
            full(1, D),
            full(D_FF, D),
            full(1, D_FF),
            full(D, D_FF),
            full(1, D),
            full(1, D),
            full(1, D),
        ],
        out_specs=pl.BlockSpec((BM, D), lambda i: (i, 0)),
        out_shape=jax.ShapeDtypeStruct((N, D), jnp.float32),
    )(
        accs[0, :N], accs[1, :N], node_feat, sel, WO, bO.reshape(1, D),
        ln1_g.reshape(1, D), ln1_b.reshape(1, D), W1, b1.reshape(1, D_FF),
        W2, b2.reshape(1, D), ln2_g.reshape(1, D), ln2_b.reshape(1, D),
    )
    return out


# speedup vs baseline: 1.2389x; 1.1354x over previous
import functools

import jax
import jax.numpy as jnp
from jax import lax
from jax.experimental import pallas as pl
from jax.experimental.pallas import tpu as pltpu
from jax.experimental.pallas import tpu_sc as plsc

N = 10000
E = 320000
D = 128
H = 8
DH = 16
D_FF = 256

NC = 2
NS = 16
NW = NC * NS

NPAD = 10240
EPW = 10240
EPAD = NW * EPW
C = 32
NCHUNK = EPW // C
ROWS_PER_SUB = NPAD // NS
ACC_W = D + H


def _dg(a, b):
    return lax.dot_general(a, b, (((1,), (1,)), ((), ())),
                           preferred_element_type=jnp.float32)


def _pack_bf16(lo, hi):
    ul = lax.bitcast_convert_type(lo, jnp.uint32)
    uh = lax.bitcast_convert_type(hi, jnp.uint32)
    bl = (ul + 0x7FFF + ((ul >> 16) & 1)) >> 16
    bh = (uh + 0x7FFF + ((uh >> 16) & 1)) >> 16
    return lax.bitcast_convert_type(bl | (bh << 16), jnp.int32)


def _qkv_body(x_ref, wql_ref, wqh_ref, wkl_ref, wkh_ref, wvl_ref, wvh_ref,
              q_ref, k_ref, v_ref):
    x = x_ref[:]
    q_ref[:] = _pack_bf16(_dg(x, wql_ref[:]), _dg(x, wqh_ref[:]))
    k_ref[:] = _pack_bf16(_dg(x, wkl_ref[:]), _dg(x, wkh_ref[:]))
    v_ref[:] = _pack_bf16(_dg(x, wvl_ref[:]), _dg(x, wvh_ref[:]))


def _edge_body(q_hbm, k_hbm, v_hbm, pk_hbm, out_hbm,
               pk_all,
               sidxA, tidxA, tshA, qbA, kbA, vbA, msgA,
               sidxB, tidxB, tshB, qbB, kbB, vbB, msgB,
               acc,
               semqA, semkA, semvA, semsA, semqB, semkB, semvB, semsB):
    cid = lax.axis_index("c")
    sid = lax.axis_index("s")
    wid = sid * NC + cid

    zeros16 = jnp.zeros((16,), jnp.float32)

    def zero_msg(m):
        def zrow(g, _):
            eids = lax.iota(jnp.int32, 16) + g * 16

            def zf(f, _):
                plsc.store_scatter(m, [eids, jnp.zeros((16,), jnp.int32) + f], zeros16)
                return 0

            lax.fori_loop(0, ACC_W, zf, 0)
            return 0

        lax.fori_loop(0, C // 16, zrow, 0)

    zero_msg(msgA)
    zero_msg(msgB)

    def zcp(i, _):
        pltpu.sync_copy(msgA, acc.at[pl.ds(sid * ROWS_PER_SUB + i * C, C)])
        return 0

    lax.fori_loop(0, ROWS_PER_SUB // C, zcp, 0)

    pltpu.sync_copy(pk_hbm.at[pl.ds(wid * EPW, EPW)], pk_all.at[pl.ds(0, EPW)])
    dummy = jnp.full((16,), N + (N << 16), jnp.int32)
    dummyn = jnp.full((16,), N, jnp.int32)
    for j in range(2 * C // 16):
        pk_all[pl.ds(EPW + j * 16, 16)] = dummy
    for j in range(C // 16):
        tshA[pl.ds(j * 16, 16)] = dummyn
        tshB[pl.ds(j * 16, 16)] = dummyn
    plsc.subcore_barrier()

    def extract_idx(ci, sidx, tidx):
        for j in range(C // 16):
            v = pk_all[pl.ds(ci * C + j * 16, 16)]
            sidx[pl.ds(j * 16, 16)] = jnp.bitwise_and(v, 0xFFFF)
            tidx[pl.ds(j * 16, 16)] = lax.shift_right_logical(v, 16)

    def gathers(sidx, tidx, qb, kb, vb, semq, semk, semv):
        return (pltpu.make_async_copy(q_hbm.at[tidx], qb, semq),
                pltpu.make_async_copy(k_hbm.at[sidx], kb, semk),
                pltpu.make_async_copy(v_hbm.at[sidx], vb, semv))

    gathersA = lambda: gathers(sidxA, tidxA, qbA, kbA, vbA, semqA, semkA, semvA)
    gathersB = lambda: gathers(sidxB, tidxB, qbB, kbB, vbB, semqB, semkB, semvB)
    scatterA = lambda: pltpu.make_async_copy(msgA, acc.at[tshA], semsA)
    scatterB = lambda: pltpu.make_async_copy(msgB, acc.at[tshB], semsB)

    def compute(qb, kb, vb, msg):
        def grp(g, _):
            eids = lax.iota(jnp.int32, 16) + g * 16
            lane = lax.iota(jnp.int32, 16)

            def hbody(h, _):
                pbase = jnp.zeros((16,), jnp.int32) + h * (DH // 2)
                hl = lane + h
                ilv = plsc.PackFormat.INTERLEAVED
                acc0 = None
                acc1 = None
                for r in range(DH // 2):
                    c = pbase + jnp.bitwise_and(hl + r, 7)
                    qi = plsc.load_gather(qb, [eids, c])
                    ki = plsc.load_gather(kb, [eids, c])
                    pb = plsc.bitcast(qi, jnp.bfloat16) * plsc.bitcast(ki, jnp.bfloat16)
                    plo, phi = plsc.unpack(pb, format=ilv)
                    acc0 = plo if acc0 is None else acc0 + plo
                    acc1 = phi if acc1 is None else acc1 + phi
                s = acc0 + acc1
                w = jnp.exp(s * 0.25)
                plsc.store_scatter(msg, [eids, jnp.zeros((16,), jnp.int32) + (D + h)], w)
                for r in range(DH // 2):
                    c = pbase + jnp.bitwise_and(hl - r, 7)
                    vi = plsc.load_gather(vb, [eids, c])
                    vlo, vhi = plsc.unpack(plsc.bitcast(vi, jnp.bfloat16), format=ilv)
                    f0 = c + pbase
                    plsc.store_scatter(msg, [eids, f0], w * vlo)
                    plsc.store_scatter(msg, [eids, f0 + 8], w * vhi)
                return 0

            lax.fori_loop(0, H, hbody, 0)
            return 0

        lax.fori_loop(0, C // 16, grp, 0)

    def shadow(tidx, tsh):
        for j in range(C // 16):
            tsh[pl.ds(j * 16, 16)] = tidx[pl.ds(j * 16, 16)]

    scatterA().start(add=True)
    scatterB().start(add=True)
    extract_idx(0, sidxA, tidxA)
    for g in gathersA():
        g.start()

    def body(i2, _):
        cA = i2 * 2
        extract_idx(cA + 1, sidxB, tidxB)
        for g in gathersB():
            g.start()
        for g in gathersA():
            g.wait()
        scatterA().wait()
        compute(qbA, kbA, vbA, msgA)
        shadow(tidxA, tshA)
        scatterA().start(add=True)
        extract_idx(cA + 2, sidxA, tidxA)
        for g in gathersA():
            g.start()
        for g in gathersB():
            g.wait()
        scatterB().wait()
        compute(qbB, kbB, vbB, msgB)
        shadow(tidxB, tshB)
        scatterB().start(add=True)
        return 0

    lax.fori_loop(0, NCHUNK // 2, body, 0)
    for g in gathersA():
        g.wait()
    scatterA().wait()
    scatterB().wait()
    plsc.subcore_barrier()
    pltpu.sync_copy(acc.at[pl.ds(sid * ROWS_PER_SUB, ROWS_PER_SUB)],
                    out_hbm.at[cid, pl.ds(sid * ROWS_PER_SUB, ROWS_PER_SUB)])


def _ln(v, g, b):
    mu = jnp.mean(v, axis=-1, keepdims=True)
    var = jnp.mean((v - mu) ** 2, axis=-1, keepdims=True)
    return (v - mu) / jnp.sqrt(var + 1e-5) * g + b


def _post_body(a0_ref, a1_ref, x_ref, sel_ref, wo_ref, bo_ref, g1_ref, be1_ref,
               w1_ref, bb1_ref, w2_ref, bb2_ref, g2_ref, be2_ref, o_ref):
    a0 = a0_ref[0]
    a1 = a1_ref[0]
    num = a0[:, :D] + a1[:, :D]
    den = a0[:, D:] + a1[:, D:]
    den = jnp.where(den == 0.0, 1.0, den)
    denf = lax.dot_general(den, sel_ref[:], (((1,), (0,)), ((), ())),
                           preferred_element_type=jnp.float32)
    att = num / denf
    x = x_ref[:]
    y = _dg(att, wo_ref[:]) + bo_ref[:] + x
    y = _ln(y, g1_ref[:], be1_ref[:])
    h1 = jnp.maximum(_dg(y, w1_ref[:]) + bb1_ref[:], 0.0)
    z = _dg(h1, w2_ref[:]) + bb2_ref[:] + y
    o_ref[:] = _ln(z, g2_ref[:], be2_ref[:])


def kernel(node_feat, edge_index, WQ, WK, WV, WO, bO, ln1_g, ln1_b,
           W1, b1, W2, b2, ln2_g, ln2_b):
    xpad = jnp.pad(node_feat, ((0, NPAD - N), (0, 0)))
    src = jnp.pad(edge_index[0].astype(jnp.int32), (0, EPAD - E), constant_values=N)
    tgt = jnp.pad(edge_index[1].astype(jnp.int32), (0, EPAD - E), constant_values=N)
    packed = jnp.bitwise_or(src, jnp.left_shift(tgt, 16))

    perm = (jnp.arange(D // 2, dtype=jnp.int32) // 8) * DH + (
        jnp.arange(D // 2, dtype=jnp.int32) % 8)
    BN = 256
    w_spec = pl.BlockSpec((D // 2, D), lambda i: (0, 0))
    qkv = pl.pallas_call(
        _qkv_body,
        grid=(NPAD // BN,),
        in_specs=[pl.BlockSpec((BN, D), lambda i: (i, 0))] + [w_spec] * 6,
        out_specs=[pl.BlockSpec((BN, D // 2), lambda i: (i, 0))] * 3,
        out_shape=[jax.ShapeDtypeStruct((NPAD, D // 2), jnp.int32)] * 3,
    )
    q, k, v = qkv(xpad, WQ[perm], WQ[perm + 8], WK[perm], WK[perm + 8],
                  WV[perm], WV[perm + 8])

    mesh = plsc.VectorSubcoreMesh(core_axis_name="c", subcore_axis_name="s",
                                  num_cores=NC, num_subcores=NS)
    half = [
        pltpu.VMEM((C,), jnp.int32),
        pltpu.VMEM((C,), jnp.int32),
        pltpu.VMEM((C,), jnp.int32),
        pltpu.VMEM((C, D // 2), jnp.int32),
        pltpu.VMEM((C, D // 2), jnp.int32),
        pltpu.VMEM((C, D // 2), jnp.int32),
        pltpu.VMEM((C, ACC_W), jnp.float32),
    ]
    edge_fn = functools.partial(
        pl.kernel,
        mesh=mesh,
        compiler_params=pltpu.CompilerParams(use_tc_tiling_on_sc=False,
                                             needs_layout_passes=False),
        out_type=jax.ShapeDtypeStruct((NC, NPAD, ACC_W), jnp.float32),
        scratch_types=(
            [pltpu.VMEM((EPW + 2 * C,), jnp.int32)]
            + half + half
            + [pltpu.VMEM_SHARED((NPAD, ACC_W), jnp.float32)]
            + [pltpu.SemaphoreType.DMA] * 8
        ),
    )(_edge_body)
    accs = edge_fn(q, k, v, packed)

    sel = (jnp.arange(D, dtype=jnp.int32)[None, :] // DH
           == jnp.arange(H, dtype=jnp.int32)[:, None]).astype(jnp.float32)
    BM = 400
    full = lambda r, c: pl.BlockSpec((r, c), lambda i: (0, 0))
    out = pl.pallas_call(
        _post_body,
        grid=(N // BM,),
        in_specs=[
            pl.BlockSpec((1, BM, ACC_W), lambda i: (0, i, 0)),
            pl.BlockSpec((1, BM, ACC_W), lambda i: (1, i, 0)),
            pl.BlockSpec((BM, D), lambda i: (i, 0)),
            full(H, D),
            full(D, D),
            full(1, D),
            full(1, D),
            full(1, D),
            full(D_FF, D),
            full(1, D_FF),
            full(D, D_FF),
            full(1, D),
            full(1, D),
            full(1, D),
        ],
        out_specs=pl.BlockSpec((BM, D), lambda i: (i, 0)),
        out_shape=jax.ShapeDtypeStruct((N, D), jnp.float32),
    )(
        accs, accs, node_feat, sel, WO, bO.reshape(1, D),
        ln1_g.reshape(1, D), ln1_b.reshape(1, D), W1, b1.reshape(1, D_FF),
        W2, b2.reshape(1, D), ln2_g.reshape(1, D), ln2_b.reshape(1, D),
    )
    return out
